# Initial kernel scaffold; baseline (speedup 1.0000x reference)
import jax
import jax.numpy as jnp
from jax.experimental import pallas as pl


def kernel(x, gate_w, gate_b, W1, b1, W2, b2):
    raise NotImplementedError("write your pallas kernel here")


# speedup vs baseline: 1.1947x; 1.1947x over previous
import functools

import jax
import jax.numpy as jnp
from jax.experimental import pallas as pl
from jax.experimental.pallas import tpu as pltpu

F = 768
S = 2048
E = 8
H = 4 * F
NH = 4
HC = H // NH


def _moe_dense_kernel(x_ref, gw_ref, gb_ref, w1_ref, b1_ref, w2_ref, b2_ref,
                      out_ref, sel_ref, h_ref):
    e = pl.program_id(0)
    j = pl.program_id(1)

    @pl.when(jnp.logical_and(e == 0, j == 0))
    def _gate():
        x = x_ref[...]
        scores = jnp.dot(x, gw_ref[...], preferred_element_type=jnp.float32)
        scores = scores + gb_ref[...]
        m = jnp.max(scores, axis=1, keepdims=True)
        ex = jnp.exp(scores - m)
        probs = ex / jnp.sum(ex, axis=1, keepdims=True)
        m1 = jnp.max(probs, axis=1, keepdims=True)
        masked = jnp.where(probs >= m1, -jnp.inf, probs)
        m2 = jnp.max(masked, axis=1, keepdims=True)
        sel = jnp.where(probs >= m2, probs, 0.0)
        sel = sel / (jnp.sum(sel, axis=1, keepdims=True) + 1e-8)
        sel_ref[...] = sel
        out_ref[...] = jnp.zeros_like(out_ref)

    x = x_ref[...]
    h = jnp.dot(x, w1_ref[0], preferred_element_type=jnp.float32) + b1_ref[0]
    h = jax.nn.gelu(h)
    y = jnp.dot(h, w2_ref[0], preferred_element_type=jnp.float32)
    emask = (jax.lax.broadcasted_iota(jnp.int32, (1, E), 1) == e)
    w = jnp.sum(jnp.where(emask, sel_ref[...], 0.0), axis=1, keepdims=True)

    @pl.when(j == 0)
    def _bias():
        out_ref[...] += w * b2_ref[0]

    out_ref[...] += w * y


@jax.jit
def _moe_dense(x2d, gate_w, gate_b2d, W1, b1, W2, b2):
    grid = (E, NH)
    return pl.pallas_call(
        _moe_dense_kernel,
        grid=grid,
        in_specs=[
            pl.BlockSpec((S, F), lambda e, j: (0, 0)),
            pl.BlockSpec((F, E), lambda e, j: (0, 0)),
            pl.BlockSpec((1, E), lambda e, j: (0, 0)),
            pl.BlockSpec((1, F, HC), lambda e, j: (e, 0, j)),
            pl.BlockSpec((1, 1, HC), lambda e, j: (e * NH + j, 0, 0)),
            pl.BlockSpec((1, HC, F), lambda e, j: (e, j, 0)),
            pl.BlockSpec((1, 1, F), lambda e, j: (e, 0, 0)),
        ],
        out_specs=pl.BlockSpec((S, F), lambda e, j: (0, 0)),
        out_shape=jax.ShapeDtypeStruct((S, F), jnp.float32),
        scratch_shapes=[
            pltpu.VMEM((S, E), jnp.float32),
            pltpu.VMEM((S, HC), jnp.float32),
        ],
    )(x2d, gate_w, gate_b2d, W1,
      b1.reshape(E * NH, 1, HC), W2, b2.reshape(E, 1, F))


def kernel(x, gate_w, gate_b, W1, b1, W2, b2):
    b, s, f = x.shape
    out = _moe_dense(x.reshape(s, f), gate_w, gate_b.reshape(1, E), W1, b1, W2, b2)
    return out.reshape(b, s, f)


# speedup vs baseline: 1.5953x; 1.3353x over previous
import functools

import jax
import jax.numpy as jnp
from jax import lax
from jax.experimental import pallas as pl
from jax.experimental.pallas import tpu as pltpu
from jax.experimental.pallas import tpu_sc as plsc

F = 768
S = 2048
E = 8
H = 4 * F
BLK = 128
NB = (2 * S) // BLK + E
NSLOT = NB * BLK
NW = 32
TCH = S // NW
FD = F // 16


def _route_kernel(x_ref, gw_ref, gb_ref, p0_ref, p1_ref, w0_ref, w1_ref,
                  eblk_ref, act_ref):
    x = x_ref[...]
    scores = jnp.dot(x, gw_ref[...], preferred_element_type=jnp.float32)
    scores = scores + gb_ref[...]
    m = jnp.max(scores, axis=1, keepdims=True)
    ex = jnp.exp(scores - m)
    probs = ex / jnp.sum(ex, axis=1, keepdims=True)

    m1 = jnp.max(probs, axis=1, keepdims=True)
    oh0 = (probs >= m1).astype(jnp.float32)
    masked = jnp.where(probs >= m1, -jnp.inf, probs)
    m2 = jnp.max(masked, axis=1, keepdims=True)
    oh1 = jnp.where(probs >= m2, 1.0, 0.0) - oh0
    sel = probs * (oh0 + oh1)
    sel = sel / (jnp.sum(sel, axis=1, keepdims=True) + 1e-8)

    w0_ref[...] = jnp.broadcast_to(
        jnp.sum(oh0 * sel, axis=1, keepdims=True), (S, 16))
    w1_ref[...] = jnp.broadcast_to(
        jnp.sum(oh1 * sel, axis=1, keepdims=True), (S, 16))

    oh = oh0 + oh1
    ti = lax.broadcasted_iota(jnp.int32, (S, S), 0)
    tj = lax.broadcasted_iota(jnp.int32, (S, S), 1)
    ltri = (tj < ti).astype(jnp.float32)
    rank = jnp.dot(ltri, oh, preferred_element_type=jnp.float32)

    counts = jnp.sum(oh, axis=0, keepdims=True)
    nb = (jnp.round(counts).astype(jnp.int32) + (BLK - 1)) // BLK
    ei = lax.broadcasted_iota(jnp.int32, (E, E), 0)
    ej = lax.broadcasted_iota(jnp.int32, (E, E), 1)
    utri = (ei < ej).astype(jnp.float32)
    gb = jnp.round(jnp.dot(nb.astype(jnp.float32), utri,
                           preferred_element_type=jnp.float32)).astype(jnp.int32)
    g = (gb * BLK).astype(jnp.float32)

    p0_ref[...] = jnp.round(
        jnp.sum(oh0 * (rank + g), axis=1, keepdims=True)).astype(jnp.int32)
    p1_ref[...] = jnp.round(
        jnp.sum(oh1 * (rank + g), axis=1, keepdims=True)).astype(jnp.int32)

    ends = (gb + nb).astype(jnp.float32)
    bk = lax.broadcasted_iota(jnp.int32, (NB, 1), 0).astype(jnp.float32)
    eblk = jnp.sum(jnp.where(ends <= bk, 1.0, 0.0), axis=1, keepdims=True)
    eblk_ref[...] = jnp.minimum(eblk, float(E - 1)).astype(jnp.int32)
    total = jnp.sum(nb.astype(jnp.float32))
    act_ref[...] = (bk < total).astype(jnp.int32)


@jax.jit
def _route(x2d, gate_w, gate_b2d):
    return pl.pallas_call(
        _route_kernel,
        in_specs=[
            pl.BlockSpec((S, F), lambda: (0, 0)),
            pl.BlockSpec((F, E), lambda: (0, 0)),
            pl.BlockSpec((1, E), lambda: (0, 0)),
        ],
        out_specs=[
            pl.BlockSpec((S, 1), lambda: (0, 0)),
            pl.BlockSpec((S, 1), lambda: (0, 0)),
            pl.BlockSpec((S, 16), lambda: (0, 0)),
            pl.BlockSpec((S, 16), lambda: (0, 0)),
            pl.BlockSpec((NB, 1), lambda: (0, 0)),
            pl.BlockSpec((NB, 1), lambda: (0, 0)),
        ],
        out_shape=[
            jax.ShapeDtypeStruct((S, 1), jnp.int32),
            jax.ShapeDtypeStruct((S, 1), jnp.int32),
            jax.ShapeDtypeStruct((S, 16), jnp.float32),
            jax.ShapeDtypeStruct((S, 16), jnp.float32),
            jax.ShapeDtypeStruct((NB, 1), jnp.int32),
            jax.ShapeDtypeStruct((NB, 1), jnp.int32),
        ],
    )(x2d, gate_w, gate_b2d)


def _dispatch_body(x_hbm, p0_hbm, p1_hbm, xs_hbm, xbuf, p0v, p1v, sem):
    wid = lax.axis_index("s") * 2 + lax.axis_index("c")
    base = wid * TCH
    pltpu.sync_copy(x_hbm.at[pl.ds(base, TCH)], xbuf)
    pltpu.sync_copy(p0_hbm.at[pl.ds(base, TCH)], p0v)
    pltpu.sync_copy(p1_hbm.at[pl.ds(base, TCH)], p1v)
    pltpu.async_copy(xbuf, xs_hbm.at[p0v], sem).wait()
    pltpu.async_copy(xbuf, xs_hbm.at[p1v], sem).wait()


@jax.jit
def _dispatch(x2d, p0, p1):
    mesh = plsc.VectorSubcoreMesh(core_axis_name="c", subcore_axis_name="s")
    return pl.kernel(
        _dispatch_body,
        out_type=jax.ShapeDtypeStruct((NSLOT, F), jnp.float32),
        mesh=mesh,
        scratch_types=[
            pltpu.VMEM((TCH, F), jnp.float32),
            pltpu.VMEM((TCH,), jnp.int32),
            pltpu.VMEM((TCH,), jnp.int32),
            pltpu.SemaphoreType.DMA,
        ],
    )(x2d, p0, p1)


def _ffn_kernel(eblk_ref, act_ref, xs_ref, w1_ref, b1_ref, w2_ref, b2_ref,
                y_ref):
    i = pl.program_id(0)

    @pl.when(act_ref[i] != 0)
    def _():
        h = jnp.dot(xs_ref[...], w1_ref[0],
                    preferred_element_type=jnp.float32) + b1_ref[0]
        h = jax.nn.gelu(h)
        y_ref[...] = jnp.dot(h, w2_ref[0],
                             preferred_element_type=jnp.float32) + b2_ref[0]


@jax.jit
def _ffn(eblk, act, xs, W1, b1, W2, b2):
    grid_spec = pltpu.PrefetchScalarGridSpec(
        num_scalar_prefetch=2,
        grid=(NB,),
        in_specs=[
            pl.BlockSpec((BLK, F), lambda i, eblk, act: (i, 0)),
            pl.BlockSpec((1, F, H), lambda i, eblk, act: (eblk[i], 0, 0)),
            pl.BlockSpec((1, 1, H), lambda i, eblk, act: (eblk[i], 0, 0)),
            pl.BlockSpec((1, H, F), lambda i, eblk, act: (eblk[i], 0, 0)),
            pl.BlockSpec((1, 1, F), lambda i, eblk, act: (eblk[i], 0, 0)),
        ],
        out_specs=pl.BlockSpec((BLK, F), lambda i, eblk, act: (i, 0)),
    )
    return pl.pallas_call(
        _ffn_kernel,
        grid_spec=grid_spec,
        out_shape=jax.ShapeDtypeStruct((NSLOT, F), jnp.float32),
    )(eblk, act, xs, W1, b1.reshape(E, 1, H), W2, b2.reshape(E, 1, F))


def _combine_body(y_hbm, p0_hbm, p1_hbm, w0_hbm, w1_hbm, out_hbm,
                  y0buf, y1buf, p0v, p1v, w0v, w1v, sem):
    wid = lax.axis_index("s") * 2 + lax.axis_index("c")
    base = wid * TCH
    pltpu.sync_copy(p0_hbm.at[pl.ds(base, TCH)], p0v)
    pltpu.sync_copy(p1_hbm.at[pl.ds(base, TCH)], p1v)
    pltpu.sync_copy(w0_hbm.at[pl.ds(base, TCH)], w0v)
    pltpu.sync_copy(w1_hbm.at[pl.ds(base, TCH)], w1v)
    pltpu.async_copy(y_hbm.at[p0v], y0buf, sem).wait()
    pltpu.async_copy(y_hbm.at[p1v], y1buf, sem).wait()

    def body(t, _):
        w0 = w0v[t, :]
        w1 = w1v[t, :]
        for d in range(FD):
            a = y0buf[t, pl.ds(d * 16, 16)]
            b = y1buf[t, pl.ds(d * 16, 16)]
            y0buf[t, pl.ds(d * 16, 16)] = w0 * a + w1 * b
        return 0

    lax.fori_loop(0, TCH, body, 0)
    pltpu.sync_copy(y0buf, out_hbm.at[pl.ds(base, TCH)])


@jax.jit
def _combine(y, p0, p1, w0, w1):
    mesh = plsc.VectorSubcoreMesh(core_axis_name="c", subcore_axis_name="s")
    return pl.kernel(
        _combine_body,
        out_type=jax.ShapeDtypeStruct((S, F), jnp.float32),
        mesh=mesh,
        scratch_types=[
            pltpu.VMEM((TCH, F), jnp.float32),
            pltpu.VMEM((TCH, F), jnp.float32),
            pltpu.VMEM((TCH,), jnp.int32),
            pltpu.VMEM((TCH,), jnp.int32),
            pltpu.VMEM((TCH, 16), jnp.float32),
            pltpu.VMEM((TCH, 16), jnp.float32),
            pltpu.SemaphoreType.DMA,
        ],
    )(y, p0, p1, w0, w1)


def kernel(x, gate_w, gate_b, W1, b1, W2, b2):
    b, s, f = x.shape
    x2d = x.reshape(s, f)
    p0, p1, w0, w1, eblk, act = _route(x2d, gate_w, gate_b.reshape(1, E))
    p0, p1 = p0.reshape(S), p1.reshape(S)
    xs = _dispatch(x2d, p0, p1)
    y = _ffn(eblk.reshape(NB), act.reshape(NB), xs, W1, b1, W2, b2)
    out = _combine(y, p0, p1, w0, w1)
    return out.reshape(b, s, f)
